# baseline (device time: 106295 ns/iter reference)
import jax
import jax.numpy as jnp
from jax import lax
from jax.experimental import pallas as pl
from jax.experimental.pallas import tpu as pltpu

N_DEV = 4
N_HOP = N_DEV - 1


def kernel(x, w_mat, scale_x, scale_w):
    m_per, k = x.shape
    _, n = w_mat.shape
    half = m_per // 2
    sub = half // 2
    NT = 16
    tk = k // NT

    def body(x_hbm, w_hbm, sx_ref, sw_ref, out_hbm,
             xstage, x8, wstage, w8, cw_buf, ccw_buf, ostage,
             xsems, wsems, osems, cw_send, cw_recv, ccw_send, ccw_recv):
        my = lax.axis_index("i")
        left = (my - 1) % N_DEV
        right = (my + 1) % N_DEV

        x_cp = lambda q, slot: pltpu.make_async_copy(
            x_hbm.at[pl.ds(q * sub, sub)], xstage.at[slot], xsems.at[slot])
        w_cp = lambda t, slot: pltpu.make_async_copy(
            w_hbm.at[pl.ds(t * tk, tk)], wstage.at[slot], wsems.at[slot])

        x_cp(0, 0).start()
        x_cp(2, 1).start()
        w_cp(0, 0).start()

        barrier_sem = pltpu.get_barrier_semaphore()
        for nbr in (left, right):
            pl.semaphore_signal(
                barrier_sem, inc=1,
                device_id=(nbr,), device_id_type=pl.DeviceIdType.MESH,
            )
        pl.semaphore_wait(barrier_sem, 2)

        w_cp(1, 1).start()
        scale = sx_ref[0] * sw_ref[0]
        rdmas = []

        def rdma(src_ref, buf, h, s, send_sems, recv_sems, dev):
            r = pltpu.make_async_remote_copy(
                src_ref=src_ref,
                dst_ref=buf.at[h, pl.ds(s * sub, sub)],
                send_sem=send_sems.at[h * 2 + s],
                recv_sem=recv_sems.at[h * 2 + s],
                device_id=(dev,), device_id_type=pl.DeviceIdType.MESH,
            )
            rdmas.append(r)
            return r

        def x_quarter(q, slot):
            x_cp(q, slot).wait()
            x8[pl.ds(q * sub, sub)] = xstage[slot].astype(jnp.float8_e4m3fn)

        x_quarter(0, 0)
        x_cp(1, 0).start()
        rdma(x8.at[pl.ds(0, sub)], cw_buf, 0, 0, cw_send, cw_recv,
             right).start()
        x_quarter(2, 1)
        x_cp(3, 1).start()
        rdma(x8.at[pl.ds(2 * sub, sub)], ccw_buf, 0, 0, ccw_send, ccw_recv,
             left).start()
        x_quarter(1, 0)
        rdma(x8.at[pl.ds(sub, sub)], cw_buf, 0, 1, cw_send, cw_recv,
             right).start()
        x_quarter(3, 1)
        rdma(x8.at[pl.ds(3 * sub, sub)], ccw_buf, 0, 1, ccw_send, ccw_recv,
             left).start()

        for t in range(NT):
            w_cp(t, t % 2).wait()
            w8[pl.ds(t * tk, tk)] = wstage[t % 2].astype(jnp.float8_e5m2)
            if t + 2 < NT:
                w_cp(t + 2, t % 2).start()

        pending = [None, None]
        acc_q = []
        epi_count = [0]

        def epi_one():
            acc, row = acc_q.pop(0)
            slot = epi_count[0] % 2
            epi_count[0] += 1
            if pending[slot] is not None:
                pending[slot].wait()
            ostage[slot] = jnp.maximum(acc * scale, 0.0)
            cp = pltpu.make_async_copy(
                ostage.at[slot], out_hbm.at[pl.ds(row, sub)], osems.at[slot])
            cp.start()
            pending[slot] = cp

        def dot_q(chunk, row):
            acc = lax.dot_general(
                chunk, w8[...],
                (((1,), (0,)), ((), ())),
                preferred_element_type=jnp.float32,
            )
            acc_q.append((acc, row))
            if len(acc_q) > 2:
                epi_one()

        def dot_own(q):
            dot_q(x8[pl.ds(q * sub, sub)], my * m_per + q * sub)

        def dot_cw(h, s):
            origin = (my - h - 1) % N_DEV
            dot_q(cw_buf[h, pl.ds(s * sub, sub)], origin * m_per + s * sub)

        def dot_ccw(h, s):
            origin = (my + h + 1) % N_DEV
            dot_q(ccw_buf[h, pl.ds(s * sub, sub)],
                  origin * m_per + half + s * sub)

        def recv_only(buf, h, s, send_sems, recv_sems, dev):
            return pltpu.make_async_remote_copy(
                src_ref=buf.at[h, pl.ds(s * sub, sub)],
                dst_ref=buf.at[h, pl.ds(s * sub, sub)],
                send_sem=send_sems.at[h * 2 + s],
                recv_sem=recv_sems.at[h * 2 + s],
                device_id=(dev,), device_id_type=pl.DeviceIdType.MESH,
            )

        def wait_and_forward(h, s):
            recv_only(cw_buf, h, s, cw_send, cw_recv, right).wait_recv()
            if h + 1 < N_HOP:
                rdma(cw_buf.at[h, pl.ds(s * sub, sub)], cw_buf, h + 1, s,
                     cw_send, cw_recv, right).start()
            recv_only(ccw_buf, h, s, ccw_send, ccw_recv, left).wait_recv()
            if h + 1 < N_HOP:
                rdma(ccw_buf.at[h, pl.ds(s * sub, sub)], ccw_buf, h + 1, s,
                     ccw_send, ccw_recv, left).start()

        wait_and_forward(0, 0)
        dot_own(0)
        dot_own(1)
        epi_one()
        wait_and_forward(0, 1)
        dot_own(2)
        epi_one()
        dot_own(3)
        epi_one()
        dot_cw(0, 0)
        epi_one()
        dot_ccw(0, 0)
        epi_one()
        wait_and_forward(1, 0)
        dot_cw(0, 1)
        epi_one()
        dot_ccw(0, 1)
        epi_one()
        wait_and_forward(1, 1)
        dot_cw(1, 0)
        epi_one()
        dot_ccw(1, 0)
        epi_one()
        dot_cw(1, 1)
        epi_one()
        dot_ccw(1, 1)
        epi_one()
        wait_and_forward(2, 0)
        dot_cw(2, 0)
        epi_one()
        dot_ccw(2, 0)
        epi_one()
        wait_and_forward(2, 1)
        dot_cw(2, 1)
        epi_one()
        dot_ccw(2, 1)
        epi_one()
        epi_one()

        for r in rdmas:
            r.wait_send()
        for cp in pending:
            cp.wait()

    return pl.pallas_call(
        body,
        out_shape=jax.ShapeDtypeStruct((N_DEV * m_per, n), jnp.float32),
        in_specs=[
            pl.BlockSpec(memory_space=pl.ANY),
            pl.BlockSpec(memory_space=pl.ANY),
            pl.BlockSpec(memory_space=pltpu.SMEM),
            pl.BlockSpec(memory_space=pltpu.SMEM),
        ],
        out_specs=pl.BlockSpec(memory_space=pl.ANY),
        scratch_shapes=[
            pltpu.VMEM((2, sub, k), jnp.float32),
            pltpu.VMEM((m_per, k), jnp.float8_e4m3fn),
            pltpu.VMEM((2, tk, n), jnp.float32),
            pltpu.VMEM((k, n), jnp.float8_e5m2),
            pltpu.VMEM((N_HOP, half, k), jnp.float8_e4m3fn),
            pltpu.VMEM((N_HOP, half, k), jnp.float8_e4m3fn),
            pltpu.VMEM((2, sub, n), jnp.float32),
            pltpu.SemaphoreType.DMA((2,)),
            pltpu.SemaphoreType.DMA((2,)),
            pltpu.SemaphoreType.DMA((2,)),
            pltpu.SemaphoreType.DMA((N_HOP * 2,)),
            pltpu.SemaphoreType.DMA((N_HOP * 2,)),
            pltpu.SemaphoreType.DMA((N_HOP * 2,)),
            pltpu.SemaphoreType.DMA((N_HOP * 2,)),
        ],
        compiler_params=pltpu.CompilerParams(
            collective_id=0, vmem_limit_bytes=100 * 1024 * 1024
        ),
    )(x, w_mat, scale_x, scale_w)
